# baseline (speedup 1.0000x reference)
import jax
import jax.numpy as jnp
from jax import lax
from jax.experimental import pallas as pl
from jax.experimental.pallas import tpu as pltpu

E = 10
TOPK = 3
B = 1024
D_IN = 1024
D_EXP = 512
T = 2
NEG = -1e30


def _mmoe_kernel(x_ref, We1_ref, be1_ref, We2_ref, be2_ref,
                 Wg_ref, Wn_ref, noise_ref, Wt1_ref, bt1_ref, Wt2_ref,
                 bt2_ref, out0_ref, out1_ref, w1buf, w2buf, sem1, sem2):
    def start_copy(e, slot):
        pltpu.make_async_copy(We1_ref.at[e], w1buf.at[slot],
                              sem1.at[slot]).start()
        pltpu.make_async_copy(We2_ref.at[e], w2buf.at[slot],
                              sem2.at[slot]).start()

    def wait_copy(e, slot):
        pltpu.make_async_copy(We1_ref.at[e], w1buf.at[slot],
                              sem1.at[slot]).wait()
        pltpu.make_async_copy(We2_ref.at[e], w2buf.at[slot],
                              sem2.at[slot]).wait()

    start_copy(0, 0)
    start_copy(1, 1)

    x = x_ref[...]

    mean = []
    sraw = []
    for i in range(T):
        mean.append(jnp.dot(x, Wg_ref[i], preferred_element_type=jnp.float32))
        sraw.append(jnp.dot(x, Wn_ref[i], preferred_element_type=jnp.float32))

    wait_copy(0, 0)
    h0 = jnp.maximum(
        jnp.dot(x, w1buf[0], preferred_element_type=jnp.float32)
        + be1_ref[0], 0.0)
    f0 = (jnp.dot(h0, w2buf[0], preferred_element_type=jnp.float32)
          + be2_ref[0])
    start_copy(2, 0)

    gsum = jnp.zeros((B, E), jnp.float32)
    iota = lax.broadcasted_iota(jnp.int32, (B, E), 1)
    for i in range(T):
        std = jax.nn.softplus(sraw[i])
        H = mean[i] + noise_ref[i] * std
        Hw = H
        for _ in range(TOPK - 1):
            m = jnp.max(Hw, axis=1, keepdims=True)
            idx = jnp.min(jnp.where(Hw == m, iota, E), axis=1, keepdims=True)
            Hw = jnp.where(iota == idx, NEG, Hw)
        thresh = jnp.max(Hw, axis=1, keepdims=True)
        Hm = jnp.where(H < thresh, NEG, H)
        mx = jnp.max(Hm, axis=1, keepdims=True)
        p = jnp.exp(Hm - mx)
        gsum = gsum + p / jnp.sum(p, axis=1, keepdims=True)

    acc = gsum[:, 0:1] * f0
    for e in range(1, E):
        slot = e % 2
        wait_copy(e, slot)
        h = jnp.maximum(
            jnp.dot(x, w1buf[slot], preferred_element_type=jnp.float32)
            + be1_ref[e], 0.0)
        f = (jnp.dot(h, w2buf[slot],
                     preferred_element_type=jnp.float32) + be2_ref[e])
        acc = acc + gsum[:, e:e + 1] * f
        if e + 2 < E:
            start_copy(e + 2, slot)

    for t, out_ref in ((0, out0_ref), (1, out1_ref)):
        ht = jnp.maximum(
            jnp.dot(acc, Wt1_ref[t],
                    preferred_element_type=jnp.float32) + bt1_ref[t], 0.0)
        out_ref[...] = (
            jnp.dot(ht, Wt2_ref[t],
                    preferred_element_type=jnp.float32) + bt2_ref[t])


@jax.jit
def kernel(x, We1, be1, We2, be2, Wg, Wn, Wt1, bt1, Wt2, bt2):
    with jax.ensure_compile_time_eval():
        nkey = jax.random.key(42)
        noise = jnp.stack([
            jax.random.normal(jax.random.fold_in(nkey, i), (B, E),
                              dtype=jnp.float32)
            for i in range(T)])

    vmem = pl.BlockSpec(memory_space=pltpu.MemorySpace.VMEM)
    hbm = pl.BlockSpec(memory_space=pl.MemorySpace.ANY)
    out0, out1 = pl.pallas_call(
        _mmoe_kernel,
        in_specs=[vmem, hbm, vmem, hbm, vmem, vmem, vmem, vmem, vmem, vmem,
                  vmem, vmem],
        out_specs=(vmem, vmem),
        out_shape=(jax.ShapeDtypeStruct((B, 256), jnp.float32),
                   jax.ShapeDtypeStruct((B, 256), jnp.float32)),
        scratch_shapes=[pltpu.VMEM((2, D_IN, D_EXP), jnp.float32),
                        pltpu.VMEM((2, D_EXP, D_EXP), jnp.float32),
                        pltpu.SemaphoreType.DMA((2,)),
                        pltpu.SemaphoreType.DMA((2,))],
    )(x, We1, be1, We2, be2, Wg, Wn, noise, Wt1, bt1, Wt2, bt2)
    return (out0, out1)


# speedup vs baseline: 1.0983x; 1.0983x over previous
import jax
import jax.numpy as jnp
from jax import lax
from jax.experimental import pallas as pl
from jax.experimental.pallas import tpu as pltpu

E = 10
TOPK = 3
B = 1024
D_IN = 1024
D_EXP = 512
T = 2
NEG = -1e30


def _mmoe_kernel(x_ref, We1_ref, be1_ref, We2_ref, be2_ref,
                 Wg_ref, Wn_ref, noise_ref, Wt1_ref, bt1_ref, Wt2_ref,
                 bt2_ref, out0_ref, out1_ref, w1buf, w2buf, sem1, sem2):
    def start_copy(e, slot):
        pltpu.make_async_copy(We1_ref.at[e], w1buf.at[slot],
                              sem1.at[slot]).start()
        pltpu.make_async_copy(We2_ref.at[e], w2buf.at[slot],
                              sem2.at[slot]).start()

    def wait_copy(e, slot):
        pltpu.make_async_copy(We1_ref.at[e], w1buf.at[slot],
                              sem1.at[slot]).wait()
        pltpu.make_async_copy(We2_ref.at[e], w2buf.at[slot],
                              sem2.at[slot]).wait()

    start_copy(0, 0)
    start_copy(1, 1)

    x = x_ref[...]

    gsum = jnp.zeros((B, E), jnp.float32)
    iota = lax.broadcasted_iota(jnp.int32, (B, E), 1)
    for i in range(T):
        mean = jnp.dot(x, Wg_ref[i], preferred_element_type=jnp.float32)
        std = jax.nn.softplus(
            jnp.dot(x, Wn_ref[i], preferred_element_type=jnp.float32))
        H = mean + noise_ref[i] * std
        Hw = H
        for _ in range(TOPK - 1):
            m = jnp.max(Hw, axis=1, keepdims=True)
            idx = jnp.min(jnp.where(Hw == m, iota, E), axis=1, keepdims=True)
            Hw = jnp.where(iota == idx, NEG, Hw)
        thresh = jnp.max(Hw, axis=1, keepdims=True)
        Hm = jnp.where(H < thresh, NEG, H)
        mx = jnp.max(Hm, axis=1, keepdims=True)
        p = jnp.exp(Hm - mx)
        gsum = gsum + p / jnp.sum(p, axis=1, keepdims=True)

    acc = jnp.zeros((B, D_EXP), jnp.float32)
    for e in range(E):
        slot = e % 2
        wait_copy(e, slot)
        h = jnp.maximum(
            jnp.dot(x, w1buf[slot], preferred_element_type=jnp.float32)
            + be1_ref[e], 0.0)
        f = (jnp.dot(h, w2buf[slot],
                     preferred_element_type=jnp.float32) + be2_ref[e])
        acc = acc + gsum[:, e:e + 1] * f
        if e + 2 < E:
            start_copy(e + 2, slot)

    for t, out_ref in ((0, out0_ref), (1, out1_ref)):
        ht = jnp.maximum(
            jnp.dot(acc, Wt1_ref[t],
                    preferred_element_type=jnp.float32) + bt1_ref[t], 0.0)
        out_ref[...] = (
            jnp.dot(ht, Wt2_ref[t],
                    preferred_element_type=jnp.float32) + bt2_ref[t])


@jax.jit
def kernel(x, We1, be1, We2, be2, Wg, Wn, Wt1, bt1, Wt2, bt2):
    with jax.ensure_compile_time_eval():
        nkey = jax.random.key(42)
        noise = jnp.stack([
            jax.random.normal(jax.random.fold_in(nkey, i), (B, E),
                              dtype=jnp.float32)
            for i in range(T)])

    vmem = pl.BlockSpec(memory_space=pltpu.MemorySpace.VMEM)
    hbm = pl.BlockSpec(memory_space=pl.MemorySpace.ANY)
    out0, out1 = pl.pallas_call(
        _mmoe_kernel,
        in_specs=[vmem, hbm, vmem, hbm, vmem, vmem, vmem, vmem, vmem, vmem,
                  vmem, vmem],
        out_specs=(vmem, vmem),
        out_shape=(jax.ShapeDtypeStruct((B, 256), jnp.float32),
                   jax.ShapeDtypeStruct((B, 256), jnp.float32)),
        scratch_shapes=[pltpu.VMEM((2, D_IN, D_EXP), jnp.float32),
                        pltpu.VMEM((2, D_EXP, D_EXP), jnp.float32),
                        pltpu.SemaphoreType.DMA((2,)),
                        pltpu.SemaphoreType.DMA((2,))],
    )(x, We1, be1, We2, be2, Wg, Wn, noise, Wt1, bt1, Wt2, bt2)
    return (out0, out1)
